# baseline (speedup 1.0000x reference)
import functools

import jax
import jax.numpy as jnp
from jax import lax
from jax.experimental import pallas as pl
from jax.experimental.pallas import tpu as pltpu
from jax.experimental.pallas import tpu_sc as plsc

_N = 10000
_E = 320000
_D = 128
_DE = 16
_H = 128
_EPS = 1e-5

_GW = 128
_BE = 6400
_BN = 1000
_NPAD = 10240
_NSUB = 16
_NCORE = 2
_NCHUNK = 2
_EC = _E // _NCHUNK


def _sc_gather(x, edge_index, chunk):
    d = x.shape[1]
    win0 = chunk * (_EC // _GW)
    mesh = plsc.VectorSubcoreMesh(core_axis_name="core", subcore_axis_name="subcore")

    @functools.partial(
        pl.kernel,
        out_type=jax.ShapeDtypeStruct((_EC, d), x.dtype),
        mesh=mesh,
    )
    def k(x_hbm, i_hbm, o_hbm):
        def body(i_vmem, o_vmem):
            pltpu.sync_copy(x_hbm.at[i_vmem.at[0]], o_vmem)

        pltpu.emit_pipeline(
            body,
            grid=(_EC // _GW,),
            in_specs=[pl.BlockSpec((1, _GW), lambda i: (0, i + win0))],
            out_specs=[pl.BlockSpec((_GW, d), lambda i: (i, 0))],
            core_axis_name=("core", "subcore"),
            dimension_semantics=(pltpu.PARALLEL,),
        )(i_hbm, o_hbm)

    return k(x, edge_index)


def _sc_scatter_add(m, dst3d, chunk):
    d = m.shape[1]
    n = _NPAD
    rows_per = n // _NSUB
    zrows = 64
    nblk = _EC // _GW
    nw = nblk // (_NCORE * _NSUB)
    ngrp = nw // 3
    nrem = nblk - nw * _NCORE * _NSUB
    blk0 = chunk * nblk
    mesh = plsc.VectorSubcoreMesh(core_axis_name="core", subcore_axis_name="subcore")

    @functools.partial(
        pl.kernel,
        out_type=jax.ShapeDtypeStruct((_NCORE, n, d), jnp.float32),
        mesh=mesh,
        scratch_types=[pltpu.VMEM_SHARED((n, d), jnp.float32),
                       pltpu.SemaphoreType.DMA,
                       pltpu.SemaphoreType.DMA,
                       pltpu.SemaphoreType.DMA],
    )
    def k(m_hbm, i_hbm, o_hbm, shared, sm, si, sa):
        cid = lax.axis_index("core")
        sid = lax.axis_index("subcore")
        wid = sid * _NCORE + cid
        row0 = sid * rows_per

        def scoped(mbuf, ibuf, zbuf):
            _scatter_body(m_hbm, i_hbm, o_hbm, shared, mbuf, ibuf, zbuf,
                          sm, si, sa, cid, wid, row0, rows_per, zrows, d,
                          nw, ngrp, nrem, blk0)

        pl.run_scoped(scoped,
                      pltpu.VMEM((2, _GW, d), jnp.float32),
                      pltpu.VMEM((2, 1, _GW), jnp.int32),
                      pltpu.VMEM((zrows, d), jnp.float32))

    return k(m, dst3d)


def _scatter_body(m_hbm, i_hbm, o_hbm, shared, mbuf, ibuf, zbuf,
                  sm, si, sa, cid, wid, row0, rows_per, zrows, d,
                  nw, ngrp, nrem, blk0):
        @pl.loop(0, zrows)
        def _(r):
            @pl.loop(0, d, step=16)
            def _(c):
                zbuf[r, pl.ds(c, 16)] = jnp.zeros((16,), jnp.float32)

        @pl.loop(0, rows_per, step=zrows)
        def _(r):
            pltpu.sync_copy(zbuf, shared.at[pl.ds(row0 + r, zrows)])

        plsc.subcore_barrier()

        base = wid * nw

        def load(w, slot):
            mrow = pl.multiple_of(w * _GW, _GW)
            pltpu.async_copy(m_hbm.at[pl.ds(mrow, _GW)], mbuf.at[slot], sm)
            pltpu.async_copy(i_hbm.at[blk0 + w], ibuf.at[slot], si)

        def wait_load(slot):
            pltpu.make_async_copy(m_hbm.at[pl.ds(0, _GW)], mbuf.at[slot], sm).wait()
            pltpu.make_async_copy(i_hbm.at[0], ibuf.at[slot], si).wait()

        def start_add(slot):
            pltpu.async_copy(mbuf.at[slot], shared.at[ibuf.at[slot, 0]], sa,
                             add=True)

        def wait_add(slot):
            pltpu.make_async_copy(mbuf.at[slot], shared.at[ibuf.at[slot, 0]],
                                  sa).wait()

        load(base, 0)
        for g in range(nw):
            s0 = g % 2
            wait_load(s0)
            start_add(s0)
            if g + 1 < nw:
                load(base + g + 1, 1 - s0)
            wait_add(s0)

        @pl.when(wid < nrem)
        def _():
            load(nw * _NCORE * _NSUB + wid, 0)
            wait_load(0)
            start_add(0)
            wait_add(0)

        plsc.subcore_barrier()

        @pl.loop(0, rows_per, step=_GW)
        def _(r):
            pltpu.sync_copy(shared.at[pl.ds(row0 + r, _GW)],
                            o_hbm.at[cid, pl.ds(row0 + r, _GW)])


def _mlp_body(xj_ref, eat_ref, w1a_ref, w1b_ref, b1_ref, w2_ref, b2_ref, m_ref):
    h = jnp.dot(xj_ref[...].astype(jnp.bfloat16), w1a_ref[...],
                preferred_element_type=jnp.float32)
    h = h + lax.dot_general(
        eat_ref[...].astype(jnp.bfloat16), w1b_ref[...],
        dimension_numbers=(((0,), (0,)), ((), ())),
        preferred_element_type=jnp.float32)
    h = jnp.maximum(h + b1_ref[...], 0.0)
    m = jnp.dot(h.astype(jnp.bfloat16), w2_ref[...],
                preferred_element_type=jnp.float32) + b2_ref[...]
    m_ref[...] = jnp.maximum(m, 0.0)


def _tc_mlp(xj, ea_t, w1a, w1b, b1, w2, b2, chunk):
    d = xj.shape[1]
    de = ea_t.shape[0]
    h = w2.shape[0]
    blk0 = chunk * (_EC // _BE)
    return pl.pallas_call(
        _mlp_body,
        grid=(_EC // _BE,),
        in_specs=[
            pl.BlockSpec((_BE, d), lambda i: (i, 0)),
            pl.BlockSpec((de, _BE), lambda i: (0, i + blk0)),
            pl.BlockSpec((d, h), lambda i: (0, 0)),
            pl.BlockSpec((de, h), lambda i: (0, 0)),
            pl.BlockSpec((1, h), lambda i: (0, 0)),
            pl.BlockSpec((h, d), lambda i: (0, 0)),
            pl.BlockSpec((1, d), lambda i: (0, 0)),
        ],
        out_specs=pl.BlockSpec((_BE, d), lambda i: (i, 0)),
        out_shape=jax.ShapeDtypeStruct((_EC, d), jnp.float32),
    )(xj, ea_t, w1a, w1b, b1, w2, b2)


def _upd_body(x_ref, p_ref, q_ref, g_ref, be_ref, w3_ref, b3_ref, o_ref):
    s = x_ref[...] + (p_ref[0] + p_ref[1]) + (q_ref[0] + q_ref[1])
    mu = jnp.mean(s, axis=1, keepdims=True)
    c = s - mu
    var = jnp.mean(c * c, axis=1, keepdims=True)
    y = c * lax.rsqrt(var + _EPS) * g_ref[...] + be_ref[...]
    y = jnp.maximum(y, 0.0)
    o_ref[...] = jnp.dot(y, w3_ref[...], preferred_element_type=jnp.float32) + b3_ref[...]


def _tc_update(x, parts, gamma, beta, w3, b3):
    n, d = x.shape
    grid = (n // _BN,)
    pspec = pl.BlockSpec((_NCORE, _BN, d), lambda i: (0, i, 0))
    return pl.pallas_call(
        _upd_body,
        grid=grid,
        in_specs=[
            pl.BlockSpec((_BN, d), lambda i: (i, 0)),
            pspec, pspec,
            pl.BlockSpec((1, d), lambda i: (0, 0)),
            pl.BlockSpec((1, d), lambda i: (0, 0)),
            pl.BlockSpec((d, d), lambda i: (0, 0)),
            pl.BlockSpec((1, d), lambda i: (0, 0)),
        ],
        out_specs=pl.BlockSpec((_BN, d), lambda i: (i, 0)),
        out_shape=jax.ShapeDtypeStruct((n, d), jnp.float32),
    )(x, *parts, gamma, beta, w3, b3)


def kernel(x, edge_index, edge_attr, W1, b1, W2, b2, gamma, beta, W3, b3):
    ea_t = edge_attr.T
    w1a = W1[:_D].astype(jnp.bfloat16)
    w1b = W1[_D:].astype(jnp.bfloat16)
    w2 = W2.astype(jnp.bfloat16)
    b1r = b1.reshape(1, _H)
    b2r = b2.reshape(1, _D)

    xjs = [None] * _NCHUNK
    ms = [None] * _NCHUNK
    parts = [None] * _NCHUNK
    dst3d = edge_index[1].reshape(_E // _GW, 1, _GW)
    xjs[0] = _sc_gather(x, edge_index, 0)
    for c in range(_NCHUNK):
        ms[c] = _tc_mlp(xjs[c], ea_t, w1a, w1b, b1r, w2, b2r, c)
        if c + 1 < _NCHUNK:
            xjs[c + 1] = _sc_gather(x, edge_index, c + 1)
        parts[c] = _sc_scatter_add(ms[c], dst3d, c)
    return _tc_update(
        x, parts,
        gamma.reshape(1, _D), beta.reshape(1, _D), W3, b3.reshape(1, _D),
    )


# speedup vs baseline: 1.0524x; 1.0524x over previous
import functools

import jax
import jax.numpy as jnp
from jax import lax
from jax.experimental import pallas as pl
from jax.experimental.pallas import tpu as pltpu
from jax.experimental.pallas import tpu_sc as plsc

_N = 10000
_E = 320000
_D = 128
_DE = 16
_H = 128
_EPS = 1e-5

_GW = 128
_BE = 6400
_BN = 1000
_NPAD = 10240
_NSUB = 16
_NCORE = 2
_NCHUNK = 2
_EC = _E // _NCHUNK


def _sc_gather(x, edge_index, chunk):
    d = x.shape[1]
    win0 = chunk * (_EC // _GW)
    mesh = plsc.VectorSubcoreMesh(core_axis_name="core", subcore_axis_name="subcore")

    @functools.partial(
        pl.kernel,
        out_type=jax.ShapeDtypeStruct((_EC, d), x.dtype),
        mesh=mesh,
    )
    def k(x_hbm, i_hbm, o_hbm):
        def body(i_vmem, o_vmem):
            pltpu.sync_copy(x_hbm.at[i_vmem.at[0]], o_vmem)

        pltpu.emit_pipeline(
            body,
            grid=(_EC // _GW,),
            in_specs=[pl.BlockSpec((1, _GW), lambda i: (0, i + win0))],
            out_specs=[pl.BlockSpec((_GW, d), lambda i: (i, 0))],
            core_axis_name=("core", "subcore"),
            dimension_semantics=(pltpu.PARALLEL,),
        )(i_hbm, o_hbm)

    return k(x, edge_index)


def _sc_scatter_add(m, dst2d, chunk):
    d = m.shape[1]
    n = _NPAD
    rows_per = n // _NSUB
    zrows = 64
    nblk = _EC // _GW
    blk0 = chunk * nblk
    mesh = plsc.VectorSubcoreMesh(core_axis_name="core", subcore_axis_name="subcore")

    @functools.partial(
        pl.kernel,
        out_type=jax.ShapeDtypeStruct((_NCORE, n, d), jnp.float32),
        mesh=mesh,
        scratch_types=[pltpu.VMEM_SHARED((n, d), jnp.float32),
                       pltpu.VMEM((zrows, d), jnp.float32)],
    )
    def k(m_hbm, i_hbm, o_hbm, shared, zbuf):
        cid = lax.axis_index("core")
        sid = lax.axis_index("subcore")
        row0 = sid * rows_per

        @pl.loop(0, zrows)
        def _(r):
            @pl.loop(0, d, step=16)
            def _(c):
                zbuf[r, pl.ds(c, 16)] = jnp.zeros((16,), jnp.float32)

        @pl.loop(0, rows_per, step=zrows)
        def _(r):
            pltpu.sync_copy(zbuf, shared.at[pl.ds(row0 + r, zrows)])

        plsc.subcore_barrier()

        def body(m_vmem, i_vmem):
            pltpu.sync_copy(m_vmem, shared.at[i_vmem.at[0]], add=True)

        pltpu.emit_pipeline(
            body,
            grid=(nblk,),
            in_specs=[
                pl.BlockSpec((_GW, d), lambda i: (i, 0)),
                pl.BlockSpec((1, _GW), lambda i: (i + blk0, 0)),
            ],
            out_specs=[],
            core_axis_name=("core", "subcore"),
            dimension_semantics=(pltpu.PARALLEL,),
        )(m_hbm, i_hbm)

        plsc.subcore_barrier()
        pltpu.sync_copy(shared.at[pl.ds(row0, rows_per)],
                        o_hbm.at[cid, pl.ds(row0, rows_per)])

    return k(m, dst2d)


def _mlp_body(xj_ref, eat_ref, w1a_ref, w1b_ref, b1_ref, w2_ref, b2_ref, m_ref):
    h = jnp.dot(xj_ref[...].astype(jnp.bfloat16), w1a_ref[...],
                preferred_element_type=jnp.float32)
    h = h + lax.dot_general(
        eat_ref[...].astype(jnp.bfloat16), w1b_ref[...],
        dimension_numbers=(((0,), (0,)), ((), ())),
        preferred_element_type=jnp.float32)
    h = jnp.maximum(h + b1_ref[...], 0.0)
    m = jnp.dot(h.astype(jnp.bfloat16), w2_ref[...],
                preferred_element_type=jnp.float32) + b2_ref[...]
    m_ref[...] = jnp.maximum(m, 0.0)


def _tc_mlp(xj, ea_t, w1a, w1b, b1, w2, b2, chunk):
    d = xj.shape[1]
    de = ea_t.shape[0]
    h = w2.shape[0]
    blk0 = chunk * (_EC // _BE)
    return pl.pallas_call(
        _mlp_body,
        grid=(_EC // _BE,),
        in_specs=[
            pl.BlockSpec((_BE, d), lambda i: (i, 0)),
            pl.BlockSpec((de, _BE), lambda i: (0, i + blk0)),
            pl.BlockSpec((d, h), lambda i: (0, 0)),
            pl.BlockSpec((de, h), lambda i: (0, 0)),
            pl.BlockSpec((1, h), lambda i: (0, 0)),
            pl.BlockSpec((h, d), lambda i: (0, 0)),
            pl.BlockSpec((1, d), lambda i: (0, 0)),
        ],
        out_specs=pl.BlockSpec((_BE, d), lambda i: (i, 0)),
        out_shape=jax.ShapeDtypeStruct((_EC, d), jnp.float32),
    )(xj, ea_t, w1a, w1b, b1, w2, b2)


def _upd_body(x_ref, p_ref, q_ref, g_ref, be_ref, w3_ref, b3_ref, o_ref):
    s = x_ref[...] + (p_ref[0] + p_ref[1]) + (q_ref[0] + q_ref[1])
    mu = jnp.mean(s, axis=1, keepdims=True)
    c = s - mu
    var = jnp.mean(c * c, axis=1, keepdims=True)
    y = c * lax.rsqrt(var + _EPS) * g_ref[...] + be_ref[...]
    y = jnp.maximum(y, 0.0)
    o_ref[...] = jnp.dot(y, w3_ref[...], preferred_element_type=jnp.float32) + b3_ref[...]


def _tc_update(x, parts, gamma, beta, w3, b3):
    n, d = x.shape
    grid = (n // _BN,)
    pspec = pl.BlockSpec((_NCORE, _BN, d), lambda i: (0, i, 0))
    return pl.pallas_call(
        _upd_body,
        grid=grid,
        in_specs=[
            pl.BlockSpec((_BN, d), lambda i: (i, 0)),
            pspec, pspec,
            pl.BlockSpec((1, d), lambda i: (0, 0)),
            pl.BlockSpec((1, d), lambda i: (0, 0)),
            pl.BlockSpec((d, d), lambda i: (0, 0)),
            pl.BlockSpec((1, d), lambda i: (0, 0)),
        ],
        out_specs=pl.BlockSpec((_BN, d), lambda i: (i, 0)),
        out_shape=jax.ShapeDtypeStruct((n, d), jnp.float32),
    )(x, *parts, gamma, beta, w3, b3)


def kernel(x, edge_index, edge_attr, W1, b1, W2, b2, gamma, beta, W3, b3):
    ea_t = edge_attr.T
    w1a = W1[:_D].astype(jnp.bfloat16)
    w1b = W1[_D:].astype(jnp.bfloat16)
    w2 = W2.astype(jnp.bfloat16)
    b1r = b1.reshape(1, _H)
    b2r = b2.reshape(1, _D)

    xjs = [None] * _NCHUNK
    ms = [None] * _NCHUNK
    parts = [None] * _NCHUNK
    dst2d = edge_index[1].reshape(_E // _GW, _GW)
    xjs[0] = _sc_gather(x, edge_index, 0)
    for c in range(_NCHUNK):
        ms[c] = _tc_mlp(xjs[c], ea_t, w1a, w1b, b1r, w2, b2r, c)
        if c + 1 < _NCHUNK:
            xjs[c + 1] = _sc_gather(x, edge_index, c + 1)
        parts[c] = _sc_scatter_add(ms[c], dst2d, c)
    return _tc_update(
        x, parts,
        gamma.reshape(1, _D), beta.reshape(1, _D), W3, b3.reshape(1, _D),
    )
